# baseline (speedup 1.0000x reference)
import functools

import jax
import jax.numpy as jnp
import numpy as np
from jax import lax
from jax.experimental import pallas as pl
from jax.experimental.pallas import tpu as pltpu
from jax.experimental.pallas import tpu_sc as plsc

_B, _C, _D, _H, _W = 2, 1, 128, 160, 160
_K = 32
_N = (5, 6, 6)
_OB = (_D // _K, _H // _K, _W // _K)
_WPB = _W // _K
_ROWS = _B * _H * _W * (_D // _K)
_CHUNK = 128
_NSEM = 8
_NC, _NS = 2, 16
_NW = _NC * _NS

_plan_cache = {}

_PERM = np.array([
    121, 35, 130, 148, 45, 176, 179, 139, 99, 144, 152, 31, 112, 85, 63,
    117, 174, 114, 82, 65, 7, 4, 101, 102, 78, 163, 157, 29, 177, 108, 83,
    129, 44, 16, 58, 123, 37, 111, 19, 61, 2, 142, 34, 156, 5, 90, 175,
    167, 110, 72, 155, 178, 153, 30, 42, 3, 70, 67, 39, 56, 169, 173, 69,
    80, 22, 6, 118, 54, 77, 147, 18, 10, 11, 53, 94, 32, 159, 15, 49, 137,
    50, 138, 20, 43, 92, 8, 140, 24, 81, 96, 154, 135, 160, 106, 128, 9,
    40, 71, 164, 93, 59, 158, 75, 131, 97, 66, 25, 73, 13, 52, 88, 62,
    150, 132, 87, 76, 60, 47, 33, 79, 14, 17, 38, 86, 23, 105, 0, 145,
    133, 41, 64, 21, 161, 166, 124, 116, 26, 165, 168, 57, 89, 146, 126,
    125, 1, 115, 28, 113, 172, 162, 48, 170, 36, 119, 151, 120, 122, 100,
    91, 55, 103, 51, 127, 98, 107, 27, 74, 136, 12, 134, 109, 84, 171,
    143, 68, 149, 141, 104, 95, 46], dtype=np.int32)


def _plan():
    if "p" in _plan_cache:
        return _plan_cache["p"]
    perm = _PERM
    hh = np.repeat(np.arange(_K), _K)
    ww = np.tile(np.arange(_K), _K)
    rows_per_batch = _H * _W * (_D // _K)
    src_l, dst_l, zdst_l = [], [], []
    for b in range(_B):
        for o0 in range(_OB[0]):
            for o1 in range(_OB[1]):
                for o2 in range(_OB[2]):
                    blk = (o0 * _N[1] + o1) * _N[2] + o2
                    s = int(perm[blk])
                    s0, r = divmod(s, _N[1] * _N[2])
                    s1, s2 = divmod(r, _N[2])
                    dst = (b * rows_per_batch
                           + ((_K * o1 + hh) * _W + _K * o2 + ww)
                           * (_D // _K) + o0)
                    if s0 < _OB[0] and s1 < _OB[1] and s2 < _OB[2]:
                        src = (b * rows_per_batch
                               + ((_K * s1 + hh) * _W + _K * s2 + ww)
                               * (_D // _K) + s0)
                        src_l.append(src)
                        dst_l.append(dst)
                    else:
                        zdst_l.append(dst)
    src = np.concatenate(src_l).astype(np.int32)
    dst = np.concatenate(dst_l).astype(np.int32)
    zdst = np.concatenate(zdst_l).astype(np.int32)

    def pad_pair(a, b):
        m = _NW * _CHUNK
        n = (-len(a)) % m
        if n:
            a = np.concatenate([a, a[:n]])
            b = np.concatenate([b, b[:n]])
        return a, b

    src, dst = pad_pair(src, dst)
    zdst, _ = pad_pair(zdst, zdst)
    kc = len(src) // (_NW * _CHUNK)
    kz = len(zdst) // (_NW * _CHUNK)
    p = (jnp.asarray(src.reshape(_NW * kc, _CHUNK)),
         jnp.asarray(dst.reshape(_NW * kc, _CHUNK)),
         jnp.asarray(zdst.reshape(_NW * kz, _CHUNK)),
         kc, kz)
    _plan_cache["p"] = p
    return p


def _make_sc_call(kc, kz):
    mesh = plsc.VectorSubcoreMesh(core_axis_name="c", subcore_axis_name="s",
                                  num_cores=_NC, num_subcores=_NS)

    @functools.partial(
        pl.kernel,
        out_type=jax.ShapeDtypeStruct((_ROWS, _K), jnp.float32),
        mesh=mesh,
        scratch_types=[
            pltpu.VMEM((kc, _CHUNK), jnp.int32),
            pltpu.VMEM((kc, _CHUNK), jnp.int32),
            pltpu.VMEM((kz, _CHUNK), jnp.int32),
            pltpu.VMEM((kc * _CHUNK, _K), jnp.float32),
            pltpu.VMEM((_CHUNK, _K), jnp.float32),
        ]
        + [pltpu.SemaphoreType.DMA] * (_NSEM + 2),
        compiler_params=pltpu.CompilerParams(use_tc_tiling_on_sc=False),
    )
    def sc_call(x_hbm, src_hbm, dst_hbm, zdst_hbm, zeros_hbm, out_hbm,
                sidx, didx, zidx, buf, zbuf, *sems):
        gsem = sems[:_NSEM]
        sem_s = sems[_NSEM]
        sem_z = sems[_NSEM + 1]
        wid = lax.axis_index("s") * _NC + lax.axis_index("c")
        pltpu.sync_copy(src_hbm.at[pl.ds(wid * kc, kc)], sidx)
        pltpu.sync_copy(dst_hbm.at[pl.ds(wid * kc, kc)], didx)
        pltpu.sync_copy(zdst_hbm.at[pl.ds(wid * kz, kz)], zidx)
        pltpu.sync_copy(zeros_hbm, zbuf)

        def zero_fire(j, carry):
            pltpu.async_copy(zbuf, out_hbm.at[zidx.at[j]], sem_z)
            return carry

        lax.fori_loop(0, kz, zero_fire, 0)

        def chunk(j):
            return buf.at[pl.ds(j * _CHUNK, _CHUNK)]

        def gfire(j, k):
            return pltpu.async_copy(x_hbm.at[sidx.at[j]], chunk(j), gsem[k])

        g = [gfire(j, j % _NSEM) for j in range(min(_NSEM, kc))]
        for j in range(kc):
            g[j].wait()
            pltpu.async_copy(chunk(j), out_hbm.at[didx.at[j]], sem_s)
            if j + _NSEM < kc:
                g.append(gfire(j + _NSEM, j % _NSEM))

        pltpu.make_async_copy(x_hbm.at[pl.ds(0, kc * _CHUNK)], buf,
                              sem_s).wait()
        pltpu.make_async_copy(x_hbm.at[pl.ds(0, kz * _CHUNK)],
                              buf.at[pl.ds(0, kz * _CHUNK)], sem_z).wait()

    return sc_call


def kernel(x):
    src3, dst3, z3, kc, kz = _plan()
    x_rows = jnp.transpose(x, (0, 1, 3, 4, 2)).reshape(_ROWS, _K)
    zeros = jnp.zeros((_CHUNK, _K), jnp.float32)
    out_rows = _make_sc_call(kc, kz)(x_rows, src3, dst3, z3, zeros)
    out = out_rows.reshape(_B, _C, _H, _W, _D)
    return jnp.transpose(out, (0, 1, 4, 2, 3))


# speedup vs baseline: 1.1139x; 1.1139x over previous
import functools

import jax
import jax.numpy as jnp
import numpy as np
from jax import lax
from jax.experimental import pallas as pl
from jax.experimental.pallas import tpu as pltpu
from jax.experimental.pallas import tpu_sc as plsc

_B, _C, _D, _H, _W = 2, 1, 128, 160, 160
_K = 32
_N = (5, 6, 6)
_OB = (_D // _K, _H // _K, _W // _K)
_WPB = _W // _K
_ROWS = _B * _H * _W * (_D // _K)
_CHUNK = 128
_NSEM = 8
_NC, _NS = 2, 16
_NW = _NC * _NS

_plan_cache = {}

_PERM = np.array([
    121, 35, 130, 148, 45, 176, 179, 139, 99, 144, 152, 31, 112, 85, 63,
    117, 174, 114, 82, 65, 7, 4, 101, 102, 78, 163, 157, 29, 177, 108, 83,
    129, 44, 16, 58, 123, 37, 111, 19, 61, 2, 142, 34, 156, 5, 90, 175,
    167, 110, 72, 155, 178, 153, 30, 42, 3, 70, 67, 39, 56, 169, 173, 69,
    80, 22, 6, 118, 54, 77, 147, 18, 10, 11, 53, 94, 32, 159, 15, 49, 137,
    50, 138, 20, 43, 92, 8, 140, 24, 81, 96, 154, 135, 160, 106, 128, 9,
    40, 71, 164, 93, 59, 158, 75, 131, 97, 66, 25, 73, 13, 52, 88, 62,
    150, 132, 87, 76, 60, 47, 33, 79, 14, 17, 38, 86, 23, 105, 0, 145,
    133, 41, 64, 21, 161, 166, 124, 116, 26, 165, 168, 57, 89, 146, 126,
    125, 1, 115, 28, 113, 172, 162, 48, 170, 36, 119, 151, 120, 122, 100,
    91, 55, 103, 51, 127, 98, 107, 27, 74, 136, 12, 134, 109, 84, 171,
    143, 68, 149, 141, 104, 95, 46], dtype=np.int32)


def _plan():
    if "p" in _plan_cache:
        return _plan_cache["p"]
    perm = _PERM
    hh = np.repeat(np.arange(_K), _K)
    ww = np.tile(np.arange(_K), _K)
    rows_per_batch = _H * _W * (_D // _K)
    src_l, dst_l, zdst_l = [], [], []
    for b in range(_B):
        for o0 in range(_OB[0]):
            for o1 in range(_OB[1]):
                for o2 in range(_OB[2]):
                    blk = (o0 * _N[1] + o1) * _N[2] + o2
                    s = int(perm[blk])
                    s0, r = divmod(s, _N[1] * _N[2])
                    s1, s2 = divmod(r, _N[2])
                    dst = (b * rows_per_batch
                           + ((_K * o1 + hh) * _W + _K * o2 + ww)
                           * (_D // _K) + o0)
                    if s0 < _OB[0] and s1 < _OB[1] and s2 < _OB[2]:
                        src = (b * rows_per_batch
                               + ((_K * s1 + hh) * _W + _K * s2 + ww)
                               * (_D // _K) + s0)
                        src_l.append(src)
                        dst_l.append(dst)
                    else:
                        zdst_l.append(dst)
    src = np.concatenate(src_l).astype(np.int32)
    dst = np.concatenate(dst_l).astype(np.int32)
    zdst = np.concatenate(zdst_l).astype(np.int32)

    def pad_pair(a, b):
        m = _NW * _CHUNK
        n = (-len(a)) % m
        if n:
            a = np.concatenate([a, a[:n]])
            b = np.concatenate([b, b[:n]])
        return a, b

    src, dst = pad_pair(src, dst)
    zdst, _ = pad_pair(zdst, zdst)
    kc = len(src) // (_NW * _CHUNK)
    kz = len(zdst) // (_NW * _CHUNK)
    p = (jnp.asarray(src.reshape(_NW, kc, _CHUNK)),
         jnp.asarray(dst.reshape(_NW, kc, _CHUNK)),
         jnp.asarray(zdst.reshape(_NW, kz, _CHUNK)),
         kc, kz)
    _plan_cache["p"] = p
    return p


def _make_sc_call(kc, kz):
    mesh = plsc.VectorSubcoreMesh(core_axis_name="c", subcore_axis_name="s",
                                  num_cores=_NC, num_subcores=_NS)

    @functools.partial(
        pl.kernel,
        out_type=jax.ShapeDtypeStruct((_ROWS, _K), jnp.float32),
        mesh=mesh,
        scratch_types=[
            pltpu.VMEM((kc, _CHUNK), jnp.int32),
            pltpu.VMEM((kc, _CHUNK), jnp.int32),
            pltpu.VMEM((kz, _CHUNK), jnp.int32),
            pltpu.VMEM((kc * _CHUNK, _K), jnp.float32),
            pltpu.VMEM((_CHUNK, _K), jnp.float32),
        ]
        + [pltpu.SemaphoreType.DMA] * (_NSEM + 2),
        compiler_params=pltpu.CompilerParams(use_tc_tiling_on_sc=False),
    )
    def sc_call(x_hbm, src_hbm, dst_hbm, zdst_hbm, zeros_hbm, out_hbm,
                sidx, didx, zidx, buf, zbuf, *sems):
        gsem = sems[:_NSEM]
        sem_s = sems[_NSEM]
        sem_z = sems[_NSEM + 1]
        wid = lax.axis_index("s") * _NC + lax.axis_index("c")
        pltpu.sync_copy(src_hbm.at[wid], sidx)
        pltpu.sync_copy(dst_hbm.at[wid], didx)
        pltpu.sync_copy(zdst_hbm.at[wid], zidx)
        pltpu.sync_copy(zeros_hbm, zbuf)

        def zero_fire(j, carry):
            pltpu.async_copy(zbuf, out_hbm.at[zidx.at[j]], sem_z)
            return carry

        lax.fori_loop(0, kz, zero_fire, 0)

        def chunk(j):
            return buf.at[pl.ds(j * _CHUNK, _CHUNK)]

        def gfire(j, k):
            return pltpu.async_copy(x_hbm.at[sidx.at[j]], chunk(j), gsem[k])

        g = [gfire(j, j % _NSEM) for j in range(min(_NSEM, kc))]
        for j in range(kc):
            g[j].wait()
            pltpu.async_copy(chunk(j), out_hbm.at[didx.at[j]], sem_s)
            if j + _NSEM < kc:
                g.append(gfire(j + _NSEM, j % _NSEM))

        pltpu.make_async_copy(x_hbm.at[pl.ds(0, kc * _CHUNK)], buf,
                              sem_s).wait()
        pltpu.make_async_copy(x_hbm.at[pl.ds(0, kz * _CHUNK)],
                              buf.at[pl.ds(0, kz * _CHUNK)], sem_z).wait()

    return sc_call


def kernel(x):
    src3, dst3, z3, kc, kz = _plan()
    x_rows = jnp.transpose(x, (0, 1, 3, 4, 2)).reshape(_ROWS, _K)
    zeros = jnp.zeros((_CHUNK, _K), jnp.float32)
    out_rows = _make_sc_call(kc, kz)(x_rows, src3, dst3, z3, zeros)
    out = out_rows.reshape(_B, _C, _H, _W, _D)
    return jnp.transpose(out, (0, 1, 4, 2, 3))


# speedup vs baseline: 1.3507x; 1.2126x over previous
import functools

import jax
import jax.numpy as jnp
import numpy as np
from jax import lax
from jax.experimental import pallas as pl
from jax.experimental.pallas import tpu as pltpu
from jax.experimental.pallas import tpu_sc as plsc

_B, _C, _D, _H, _W = 2, 1, 128, 160, 160
_K = 32
_KQ = 8
_N = (5, 6, 6)
_OB = (_D // _K, _H // _K, _W // _K)
_Q = _D // _K
_NC, _NS = 2, 16
_NW = _NC * _NS

_plan_cache = {}

_PERM = np.array([
    121, 35, 130, 148, 45, 176, 179, 139, 99, 144, 152, 31, 112, 85, 63,
    117, 174, 114, 82, 65, 7, 4, 101, 102, 78, 163, 157, 29, 177, 108, 83,
    129, 44, 16, 58, 123, 37, 111, 19, 61, 2, 142, 34, 156, 5, 90, 175,
    167, 110, 72, 155, 178, 153, 30, 42, 3, 70, 67, 39, 56, 169, 173, 69,
    80, 22, 6, 118, 54, 77, 147, 18, 10, 11, 53, 94, 32, 159, 15, 49, 137,
    50, 138, 20, 43, 92, 8, 140, 24, 81, 96, 154, 135, 160, 106, 128, 9,
    40, 71, 164, 93, 59, 158, 75, 131, 97, 66, 25, 73, 13, 52, 88, 62,
    150, 132, 87, 76, 60, 47, 33, 79, 14, 17, 38, 86, 23, 105, 0, 145,
    133, 41, 64, 21, 161, 166, 124, 116, 26, 165, 168, 57, 89, 146, 126,
    125, 1, 115, 28, 113, 172, 162, 48, 170, 36, 119, 151, 120, 122, 100,
    91, 55, 103, 51, 127, 98, 107, 27, 74, 136, 12, 134, 109, 84, 171,
    143, 68, 149, 141, 104, 95, 46], dtype=np.int32)


def _plan():
    if "p" in _plan_cache:
        return _plan_cache["p"]
    copy_items, zero_items = [], []
    for b in range(_B):
        for o0 in range(_OB[0]):
            for o1 in range(_OB[1]):
                for o2 in range(_OB[2]):
                    blk = (o0 * _N[1] + o1) * _N[2] + o2
                    s = int(_PERM[blk])
                    s0, r = divmod(s, _N[1] * _N[2])
                    s1, s2 = divmod(r, _N[2])
                    for qb in range(_K // _KQ):
                        if s0 < _OB[0] and s1 < _OB[1] and s2 < _OB[2]:
                            copy_items.append(
                                (b, _K * s1 + _KQ * qb, _K * s2, _K * s0,
                                 _K * o1 + _KQ * qb, _K * o2, _K * o0)
                                + (0,) * 9)
                        else:
                            zero_items.append(
                                (b, _K * o1 + _KQ * qb, _K * o2, _K * o0)
                                + (0,) * 12)

    def pad_items(items):
        n = (-len(items)) % _NW
        return items + items[:n]

    copy_items = pad_items(copy_items)
    zero_items = pad_items(zero_items)
    kc = len(copy_items) // _NW
    kz = len(zero_items) // _NW
    ctab = (np.array(copy_items, np.int32)
            .reshape(kc, _NW, 16).transpose(1, 0, 2).copy())
    ztab = (np.array(zero_items, np.int32)
            .reshape(kz, _NW, 16).transpose(1, 0, 2).copy())
    p = (jnp.asarray(ctab), jnp.asarray(ztab), kc, kz)
    _plan_cache["p"] = p
    return p


def _make_sc_call(kc, kz):
    mesh = plsc.VectorSubcoreMesh(core_axis_name="c", subcore_axis_name="s",
                                  num_cores=_NC, num_subcores=_NS)

    @functools.partial(
        pl.kernel,
        out_type=jax.ShapeDtypeStruct((_B, _H, _W, _D), jnp.float32),
        mesh=mesh,
        scratch_types=(
            [pltpu.VMEM((kc, 16), jnp.int32),
             pltpu.VMEM((kz, 16), jnp.int32),
             pltpu.VMEM((kc, _KQ, _K, _K), jnp.float32),
             pltpu.VMEM((_KQ, _K, _K), jnp.float32)]
            + [pltpu.SemaphoreType.DMA] * (kc + 2)
        ),
        compiler_params=pltpu.CompilerParams(use_tc_tiling_on_sc=False),
    )
    def sc_call(x_hbm, ctab_hbm, ztab_hbm, zeros_hbm, out_hbm,
                ctab, ztab, buf, zbuf, *sems):
        gsem = sems[:kc]
        sem_s = sems[kc]
        sem_z = sems[kc + 1]
        wid = lax.axis_index("s") * _NC + lax.axis_index("c")
        pltpu.sync_copy(ctab_hbm.at[wid], ctab)
        pltpu.sync_copy(ztab_hbm.at[wid], ztab)
        zcp = pltpu.async_copy(zeros_hbm, zbuf, sem_z)

        def src_at(r):
            return x_hbm.at[r[0], pl.ds(pl.multiple_of(r[1], _KQ), _KQ),
                            pl.ds(pl.multiple_of(r[2], _K), _K),
                            pl.ds(pl.multiple_of(r[3], _K), _K)]

        def dst_at(r):
            return out_hbm.at[r[0], pl.ds(pl.multiple_of(r[4], _KQ), _KQ),
                              pl.ds(pl.multiple_of(r[5], _K), _K),
                              pl.ds(pl.multiple_of(r[6], _K), _K)]

        def zdst_at(r):
            return out_hbm.at[r[0], pl.ds(pl.multiple_of(r[1], _KQ), _KQ),
                              pl.ds(pl.multiple_of(r[2], _K), _K),
                              pl.ds(pl.multiple_of(r[3], _K), _K)]

        rows = [ctab[i, :] for i in range(kc)]
        g = [pltpu.async_copy(src_at(rows[i]), buf.at[i], gsem[i])
             for i in range(kc)]

        zcp.wait()
        for i in range(kz):
            pltpu.async_copy(zbuf, zdst_at(ztab[i, :]), sem_z)

        for i in range(kc):
            g[i].wait()
            pltpu.async_copy(buf.at[i], dst_at(rows[i]), sem_s)

        for i in range(kc):
            pltpu.make_async_copy(
                x_hbm.at[0, pl.ds(0, _KQ), pl.ds(0, _K), pl.ds(0, _K)],
                buf.at[i], sem_s).wait()
        for i in range(kz):
            pltpu.make_async_copy(
                x_hbm.at[0, pl.ds(0, _KQ), pl.ds(0, _K), pl.ds(0, _K)],
                zbuf, sem_z).wait()

    return sc_call


def kernel(x):
    ctab, ztab, kc, kz = _plan()
    x4 = jnp.transpose(x, (0, 1, 3, 4, 2)).reshape(_B, _H, _W, _D)
    zeros = jnp.zeros((_KQ, _K, _K), jnp.float32)
    out4 = _make_sc_call(kc, kz)(x4, ctab, ztab, zeros)
    out = out4.reshape(_B, _C, _H, _W, _D)
    return jnp.transpose(out, (0, 1, 4, 2, 3))


# speedup vs baseline: 1.3517x; 1.0007x over previous
import functools

import jax
import jax.numpy as jnp
import numpy as np
from jax import lax
from jax.experimental import pallas as pl
from jax.experimental.pallas import tpu as pltpu
from jax.experimental.pallas import tpu_sc as plsc

_B, _C, _D, _H, _W = 2, 1, 128, 160, 160
_K = 32
_KQ = 8
_N = (5, 6, 6)
_OB = (_D // _K, _H // _K, _W // _K)
_Q = _D // _K
_NC, _NS = 2, 16
_NW = _NC * _NS

_plan_cache = {}

_PERM = np.array([
    121, 35, 130, 148, 45, 176, 179, 139, 99, 144, 152, 31, 112, 85, 63,
    117, 174, 114, 82, 65, 7, 4, 101, 102, 78, 163, 157, 29, 177, 108, 83,
    129, 44, 16, 58, 123, 37, 111, 19, 61, 2, 142, 34, 156, 5, 90, 175,
    167, 110, 72, 155, 178, 153, 30, 42, 3, 70, 67, 39, 56, 169, 173, 69,
    80, 22, 6, 118, 54, 77, 147, 18, 10, 11, 53, 94, 32, 159, 15, 49, 137,
    50, 138, 20, 43, 92, 8, 140, 24, 81, 96, 154, 135, 160, 106, 128, 9,
    40, 71, 164, 93, 59, 158, 75, 131, 97, 66, 25, 73, 13, 52, 88, 62,
    150, 132, 87, 76, 60, 47, 33, 79, 14, 17, 38, 86, 23, 105, 0, 145,
    133, 41, 64, 21, 161, 166, 124, 116, 26, 165, 168, 57, 89, 146, 126,
    125, 1, 115, 28, 113, 172, 162, 48, 170, 36, 119, 151, 120, 122, 100,
    91, 55, 103, 51, 127, 98, 107, 27, 74, 136, 12, 134, 109, 84, 171,
    143, 68, 149, 141, 104, 95, 46], dtype=np.int32)


def _plan():
    if "p" in _plan_cache:
        return _plan_cache["p"]
    copy_items, zero_items = [], []
    for b in range(_B):
        for o0 in range(_OB[0]):
            for o1 in range(_OB[1]):
                for o2 in range(_OB[2]):
                    blk = (o0 * _N[1] + o1) * _N[2] + o2
                    s = int(_PERM[blk])
                    s0, r = divmod(s, _N[1] * _N[2])
                    s1, s2 = divmod(r, _N[2])
                    for qb in range(_K // _KQ):
                        if s0 < _OB[0] and s1 < _OB[1] and s2 < _OB[2]:
                            copy_items.append(
                                (b, _K * s1 + _KQ * qb, _K * s2, _K * s0,
                                 _K * o1 + _KQ * qb, _K * o2, _K * o0)
                                + (0,) * 9)
                        else:
                            zero_items.append(
                                (b, _K * o1 + _KQ * qb, _K * o2, _K * o0)
                                + (0,) * 12)

    def pad_items(items):
        n = (-len(items)) % _NW
        return items + items[:n]

    copy_items = pad_items(copy_items)
    zero_items = pad_items(zero_items)
    kc = len(copy_items) // _NW
    kz = len(zero_items) // _NW
    ctab = (np.array(copy_items, np.int32)
            .reshape(kc, _NW, 16).transpose(1, 0, 2))
    ztab = (np.array(zero_items, np.int32)
            .reshape(kz, _NW, 16).transpose(1, 0, 2))
    tab = np.concatenate([ctab, ztab], axis=1).copy()
    p = (jnp.asarray(tab), kc, kz)
    _plan_cache["p"] = p
    return p


def _make_sc_call(kc, kz):
    mesh = plsc.VectorSubcoreMesh(core_axis_name="c", subcore_axis_name="s",
                                  num_cores=_NC, num_subcores=_NS)

    @functools.partial(
        pl.kernel,
        out_type=jax.ShapeDtypeStruct((_B, _H, _W, _D), jnp.float32),
        mesh=mesh,
        scratch_types=(
            [pltpu.VMEM((kc + kz, 16), jnp.int32),
             pltpu.VMEM((kc, _KQ, _K, _K), jnp.float32),
             pltpu.VMEM((_KQ, _K, _K), jnp.float32)]
            + [pltpu.SemaphoreType.DMA] * (kc + 2)
        ),
        compiler_params=pltpu.CompilerParams(use_tc_tiling_on_sc=False),
    )
    def sc_call(x_hbm, tab_hbm, zeros_hbm, out_hbm,
                tab, buf, zbuf, *sems):
        gsem = sems[:kc]
        sem_s = sems[kc]
        sem_z = sems[kc + 1]
        wid = lax.axis_index("s") * _NC + lax.axis_index("c")
        zcp = pltpu.async_copy(zeros_hbm, zbuf, sem_z)
        pltpu.sync_copy(tab_hbm.at[wid], tab)

        def src_at(r):
            return x_hbm.at[r[0], pl.ds(pl.multiple_of(r[1], _KQ), _KQ),
                            pl.ds(pl.multiple_of(r[2], _K), _K),
                            pl.ds(pl.multiple_of(r[3], _K), _K)]

        def dst_at(r):
            return out_hbm.at[r[0], pl.ds(pl.multiple_of(r[4], _KQ), _KQ),
                              pl.ds(pl.multiple_of(r[5], _K), _K),
                              pl.ds(pl.multiple_of(r[6], _K), _K)]

        def zdst_at(r):
            return out_hbm.at[r[0], pl.ds(pl.multiple_of(r[1], _KQ), _KQ),
                              pl.ds(pl.multiple_of(r[2], _K), _K),
                              pl.ds(pl.multiple_of(r[3], _K), _K)]

        rows = [tab[i, :] for i in range(kc)]
        zrows = [tab[kc + i, :] for i in range(kz)]
        zcp.wait()
        g = []
        for i in range(max(kc, kz)):
            if i < kc:
                g.append(pltpu.async_copy(src_at(rows[i]), buf.at[i],
                                          gsem[i]))
            if i < kz:
                pltpu.async_copy(zbuf, zdst_at(zrows[i]), sem_z)

        for i in range(kc):
            g[i].wait()
            pltpu.async_copy(buf.at[i], dst_at(rows[i]), sem_s)

        for i in range(kc):
            pltpu.make_async_copy(
                x_hbm.at[0, pl.ds(0, _KQ), pl.ds(0, _K), pl.ds(0, _K)],
                buf.at[i], sem_s).wait()
        for i in range(kz):
            pltpu.make_async_copy(
                x_hbm.at[0, pl.ds(0, _KQ), pl.ds(0, _K), pl.ds(0, _K)],
                zbuf, sem_z).wait()

    return sc_call


def kernel(x):
    tab, kc, kz = _plan()
    x4 = jnp.transpose(x, (0, 1, 3, 4, 2)).reshape(_B, _H, _W, _D)
    zeros = jnp.zeros((_KQ, _K, _K), jnp.float32)
    out4 = _make_sc_call(kc, kz)(x4, tab, zeros)
    out = out4.reshape(_B, _C, _H, _W, _D)
    return jnp.transpose(out, (0, 1, 4, 2, 3))
